# baseline (speedup 1.0000x reference)
import jax
import jax.numpy as jnp
from jax.experimental import pallas as pl
from jax.experimental.pallas import tpu as pltpu

_ZETA = 1.1
_GAMMA = -0.1
_DROP_RATIO = 0.5
_DROP_THRESHOLD_U32 = int(_DROP_RATIO * (1 << 32))


def _conv_kernel(x_ref, w_ref, mask_ref, s_ref, qmin_ref, qmax_ref,
                 bias_ref, rand_ref, acti_ref, out_ref):
    _, hp, wp, c = x_ref.shape
    ho, wo = hp - 2, wp - 2
    mo = ho * wo
    o = out_ref.shape[-1]

    s = s_ref[...]
    hmask = jnp.clip((_ZETA - _GAMMA) * jax.nn.sigmoid(mask_ref[...]) + _GAMMA,
                     0.0, 1.0)
    qw = jnp.floor(w_ref[...] * (1.0 / s)) + hmask
    qw = jnp.clip(qw, qmin_ref[...], qmax_ref[...]) * s

    xb = x_ref[0]
    acc = jnp.zeros((mo, o), jnp.float32)
    for i in range(3):
        xcat = jnp.concatenate(
            [xb[i:i + ho, j:j + wo, :].reshape(mo, c) for j in range(3)], axis=1)
        wt = qw[i * 3 * c:(i + 1) * 3 * c, :]
        acc = acc + jnp.dot(xcat, wt, preferred_element_type=jnp.float32)

    acc = acc + bias_ref[...]
    acc = jnp.maximum(acc, 0.0)

    q = jnp.round(acc * acti_ref[0])
    q = jnp.clip(q, acti_ref[2], acti_ref[3]) * acti_ref[1]

    keep = rand_ref[...] < jnp.uint32(_DROP_THRESHOLD_U32)
    out_ref[...] = jnp.where(keep, q, acc)


@jax.jit
def _adaq_conv(x, weight, bias, round_mask, w_scale, w_qmin, w_qmax,
               a_scale, a_qmin, a_qmax, rng_key):
    n, c, h, w = x.shape
    o, _, kh, kw = weight.shape
    ho, wo = h, w
    m = n * ho * wo
    mo = ho * wo

    xp = jnp.pad(x.transpose(0, 2, 3, 1), ((0, 0), (1, 1), (1, 1), (0, 0)))

    wt = weight.transpose(2, 3, 1, 0).reshape(kh * kw * c, o)
    mt = round_mask.transpose(2, 3, 1, 0).reshape(kh * kw * c, o)
    s_row = jnp.maximum(w_scale, 1e-8).reshape(1, o)
    qmin_row = w_qmin.reshape(1, o)
    qmax_row = w_qmax.reshape(1, o)
    bias_row = bias.reshape(1, o)

    rand_bits = jax.random.bits(jax.random.wrap_key_data(rng_key), (m, o),
                                dtype=jnp.uint32)

    a_scale_f = a_scale.reshape(()).astype(jnp.float32)
    acti = jnp.stack([
        1.0 / a_scale_f,
        a_scale_f,
        a_qmin.reshape(()).astype(jnp.float32),
        a_qmax.reshape(()).astype(jnp.float32),
    ])

    out2d = pl.pallas_call(
        _conv_kernel,
        out_shape=jax.ShapeDtypeStruct((m, o), jnp.float32),
        grid=(n,),
        in_specs=[
            pl.BlockSpec((1, h + 2, w + 2, c), lambda i: (i, 0, 0, 0)),
            pl.BlockSpec((kh * kw * c, o), lambda i: (0, 0)),
            pl.BlockSpec((kh * kw * c, o), lambda i: (0, 0)),
            pl.BlockSpec((1, o), lambda i: (0, 0)),
            pl.BlockSpec((1, o), lambda i: (0, 0)),
            pl.BlockSpec((1, o), lambda i: (0, 0)),
            pl.BlockSpec((1, o), lambda i: (0, 0)),
            pl.BlockSpec((mo, o), lambda i: (i, 0)),
            pl.BlockSpec(memory_space=pltpu.MemorySpace.SMEM),
        ],
        out_specs=pl.BlockSpec((mo, o), lambda i: (i, 0)),
        compiler_params=pltpu.CompilerParams(
            dimension_semantics=("parallel",),
            vmem_limit_bytes=48 * 1024 * 1024,
        ),
    )(xp, wt, mt, s_row, qmin_row, qmax_row, bias_row, rand_bits, acti)

    return out2d.reshape(n, ho, wo, o).transpose(0, 3, 1, 2)


def kernel(x, weight, bias, round_mask, w_scale, w_qmin, w_qmax,
           a_scale, a_qmin, a_qmax, rng_key):
    return _adaq_conv(x, weight, bias, round_mask, w_scale, w_qmin, w_qmax,
                      a_scale, a_qmin, a_qmax, rng_key)
